# baseline (speedup 1.0000x reference)
import functools

import jax
import jax.numpy as jnp
from jax import lax
from jax.experimental import pallas as pl
from jax.experimental.pallas import tpu as pltpu
from jax.experimental.pallas import tpu_sc as plsc

B = 16384
D = 64
W = 128
NE = 1000000
NR = 1000
NC = 2
NS = 16
NW = NC * NS
BPW = B // NW
CH = 128
NCH = BPW // CH
L = 16

_mesh = plsc.VectorSubcoreMesh(
    core_axis_name="c", subcore_axis_name="s", num_cores=NC, num_subcores=NS
)


def _lookup(idx2d, tabp, out, wid, idxb, offb, pbuf, obuf, gsem, wsem):
    row0 = wid * NCH
    pltpu.sync_copy(idx2d.at[pl.ds(row0, NCH)], idxb)
    for g in range(NCH * CH // L):
        v = idxb[g // (CH // L), pl.ds((g % (CH // L)) * L, L)]
        idxb[g // (CH // L), pl.ds((g % (CH // L)) * L, L)] = (
            jax.lax.shift_right_logical(v, 1)
        )
        offb[pl.ds(g * L, L)] = jax.lax.bitwise_and(v, 1) * D
    base = wid * BPW

    def chunk(c, carry):
        pltpu.async_copy(tabp.at[idxb.at[c]], pbuf, gsem).wait()
        for g in range(CH // L):
            ovec = offb[pl.ds(c * CH + g * L, L)]
            for r in range(L):
                j = g * L + r
                off = ovec[r]
                for cc in range(D // L):
                    obuf[j, pl.ds(cc * L, L)] = (
                        pbuf[j, pl.ds(off + cc * L, L)]
                    )
        pltpu.async_copy(obuf, out.at[pl.ds(base + c * CH, CH)], wsem).wait()
        return carry

    lax.fori_loop(0, NCH, chunk, 0, unroll=False)


@functools.partial(
    pl.kernel,
    out_type=(
        jax.ShapeDtypeStruct((B, D), jnp.float32),
        jax.ShapeDtypeStruct((B, D), jnp.float32),
        jax.ShapeDtypeStruct((B, D), jnp.float32),
    ),
    mesh=_mesh,
    scratch_types=[
        pltpu.VMEM((NCH, CH), jnp.int32),
        pltpu.VMEM((BPW,), jnp.int32),
        pltpu.VMEM((CH, W), jnp.float32),
        pltpu.VMEM((CH, D), jnp.float32),
        pltpu.SemaphoreType.DMA,
        pltpu.SemaphoreType.DMA,
    ],
)
def _gather(hidx, ridx, tidx, etabp, rtabp,
            out_h, out_r, out_t,
            idxb, offb, pbuf, obuf, gsem, wsem):
    wid = lax.axis_index("s") * NC + lax.axis_index("c")
    _lookup(hidx, etabp, out_h, wid, idxb, offb, pbuf, obuf, gsem, wsem)
    _lookup(tidx, etabp, out_t, wid, idxb, offb, pbuf, obuf, gsem, wsem)
    _lookup(ridx, rtabp, out_r, wid, idxb, offb, pbuf, obuf, gsem, wsem)


def kernel(head, relation, tail, entity_table, relation_table):
    h = head.astype(jnp.int32).reshape(B // CH, CH)
    r = relation.astype(jnp.int32).reshape(B // CH, CH)
    t = tail.astype(jnp.int32).reshape(B // CH, CH)
    etabp = entity_table.reshape(NE // 2, W)
    rtabp = relation_table.reshape(NR // 2, W)
    return _gather(h, r, t, etabp, rtabp)


# speedup vs baseline: 1.7463x; 1.7463x over previous
import functools

import jax
import jax.numpy as jnp
from jax import lax
from jax.experimental import pallas as pl
from jax.experimental.pallas import tpu as pltpu
from jax.experimental.pallas import tpu_sc as plsc

B = 16384
D = 64
NR = 1000
NC = 2
NS = 16
NW = NC * NS
BPW = B // NW
L = 16

_mesh = plsc.VectorSubcoreMesh(
    core_axis_name="c", subcore_axis_name="s", num_cores=NC, num_subcores=NS
)


def _entity_lookup(idx_hbm, tab, out, base, idxb, rows, sem):
    pltpu.sync_copy(idx_hbm.at[pl.ds(base, BPW)], idxb)

    def group(g, carry):
        svec = idxb[pl.ds(g * L, L)]
        for r in range(L):
            i = svec[r]
            pltpu.async_copy(
                tab.at[pl.ds(i, 1)],
                rows.at[pl.ds(g * L + r, 1)],
                sem,
            )
        return carry

    lax.fori_loop(0, BPW // L, group, 0, unroll=False)
    pltpu.make_async_copy(tab.at[pl.ds(0, BPW)], rows, sem).wait()
    pltpu.sync_copy(rows, out.at[pl.ds(base, BPW)])


CH = 128


def _relation_lookup(idx_hbm, rtab, out, base, sid, idxb, relv, rows, sem):
    pltpu.sync_copy(idx_hbm.at[pl.ds(base, BPW)], idxb)

    @pl.when(sid == 0)
    def _():
        pltpu.sync_copy(rtab, relv)

    plsc.subcore_barrier()
    copies = []
    for c in range(BPW // CH):
        copies.append(
            pltpu.async_copy(
                relv.at[idxb.at[pl.ds(c * CH, CH)]],
                rows.at[pl.ds(c * CH, CH)],
                sem,
            )
        )
    for cp in copies:
        cp.wait()
    pltpu.sync_copy(rows, out.at[pl.ds(base, BPW)])


@functools.partial(
    pl.kernel,
    out_type=(
        jax.ShapeDtypeStruct((B, D), jnp.float32),
        jax.ShapeDtypeStruct((B, D), jnp.float32),
        jax.ShapeDtypeStruct((B, D), jnp.float32),
    ),
    mesh=_mesh,
    scratch_types=[
        pltpu.VMEM((BPW,), jnp.int32),
        pltpu.VMEM_SHARED((NR, D), jnp.float32),
        pltpu.VMEM((BPW, D), jnp.float32),
        pltpu.SemaphoreType.DMA,
    ],
)
def _sc_gather(head_hbm, rel_hbm, tail_hbm, etab, rtab,
               out_h, out_r, out_t,
               idxb, relv, rows, sem):
    sid = lax.axis_index("s")
    wid = sid * NC + lax.axis_index("c")
    base = wid * BPW
    _entity_lookup(head_hbm, etab, out_h, base, idxb, rows, sem)
    _entity_lookup(tail_hbm, etab, out_t, base, idxb, rows, sem)
    _relation_lookup(rel_hbm, rtab, out_r, base, sid, idxb, relv, rows, sem)


def kernel(head, relation, tail, entity_table, relation_table):
    h = head.astype(jnp.int32)
    r = relation.astype(jnp.int32)
    t = tail.astype(jnp.int32)
    return _sc_gather(h, r, t, entity_table, relation_table)
